# baseline (speedup 1.0000x reference)
import functools

import jax
import jax.numpy as jnp
from jax import lax
from jax.experimental import pallas as pl
from jax.experimental.pallas import tpu as pltpu
from jax.experimental.pallas import tpu_sc as plsc

B = 16384
D = 32
H = 64
W = 128
PACK = W // D
NC = 2
NS = 16
NW = NC * NS
ROWS = B // NW
L = 16


def _sc_gather_body(uid_hbm, pid_hbm, iid_hbm, ut_hbm, pt_hbm, it_hbm,
                    u_out, p_out, i_out, idx_v, idx2_v, rows_v, sem):
    wid = lax.axis_index("s") * NC + lax.axis_index("c")
    base = wid * ROWS
    for ids_hbm, tab_hbm, out_hbm in ((uid_hbm, ut_hbm, u_out),
                                      (pid_hbm, pt_hbm, p_out),
                                      (iid_hbm, it_hbm, i_out)):
        pltpu.sync_copy(ids_hbm.at[pl.ds(base, ROWS)], idx_v)
        for k in range(ROWS // L):
            sl = pl.ds(k * L, L)
            idx2_v[sl] = lax.shift_right_logical(idx_v[sl], 2)
        pltpu.async_copy(tab_hbm.at[idx2_v], rows_v, sem).wait()
        pltpu.sync_copy(rows_v, out_hbm.at[pl.ds(base, ROWS)])


_sc_gather = pl.kernel(
    _sc_gather_body,
    out_type=(
        jax.ShapeDtypeStruct((B, W), jnp.float32),
        jax.ShapeDtypeStruct((B, W), jnp.float32),
        jax.ShapeDtypeStruct((B, W), jnp.float32),
    ),
    mesh=plsc.VectorSubcoreMesh(core_axis_name="c", subcore_axis_name="s"),
    scratch_types=[
        pltpu.VMEM((ROWS,), jnp.int32),
        pltpu.VMEM((ROWS,), jnp.int32),
        pltpu.VMEM((ROWS, W), jnp.float32),
        pltpu.SemaphoreType.DMA,
    ],
)


BS = 2048


def _mlp_body(u_ref, p_ref, i_ref, uid_ref, pid_ref, iid_ref,
              w1u_ref, w1p_ref, w1i_ref, b1_ref, w2_ref, b2_ref, out_ref):
    lane_blk = lax.broadcasted_iota(jnp.int32, (BS, W), 1) >> 5

    def pick(x_ref, id_ref):
        sel = lane_blk == (id_ref[...] & (PACK - 1))
        return jnp.where(sel, x_ref[...], 0.0)

    h = (jnp.dot(pick(u_ref, uid_ref), w1u_ref[...],
                 preferred_element_type=jnp.float32)
         + jnp.dot(pick(p_ref, pid_ref), w1p_ref[...],
                   preferred_element_type=jnp.float32)
         + jnp.dot(pick(i_ref, iid_ref), w1i_ref[...],
                   preferred_element_type=jnp.float32)
         + b1_ref[...])
    h = jnp.maximum(h, 0.0)
    out_ref[...] = (jnp.dot(h, w2_ref[...], preferred_element_type=jnp.float32)
                    + b2_ref[...])


def _tc_mlp(u, p, i, uid, pid, iid, w1u4, w1p4, w1i4, b1, w2, b2):
    grid = (B // BS,)
    emb_spec = pl.BlockSpec((BS, W), lambda j: (j, 0))
    id_spec = pl.BlockSpec((BS, 1), lambda j: (j, 0))
    full = lambda shape: pl.BlockSpec(shape, lambda j: (0, 0))
    return pl.pallas_call(
        _mlp_body,
        grid=grid,
        in_specs=[emb_spec, emb_spec, emb_spec, id_spec, id_spec, id_spec,
                  full((W, H)), full((W, H)), full((W, H)), full((1, H)),
                  full((H, 1)), full((1, 1))],
        out_specs=pl.BlockSpec((BS, 1), lambda j: (j, 0)),
        out_shape=jax.ShapeDtypeStruct((B, 1), jnp.float32),
    )(u, p, i, uid, pid, iid, w1u4, w1p4, w1i4, b1, w2, b2)


def kernel(user_ids, product_ids, interaction_ids, user_table, product_table,
           interaction_table, W1, b1, W2, b2):
    uid = user_ids.astype(jnp.int32)
    pid = product_ids.astype(jnp.int32)
    iid = interaction_ids.astype(jnp.int32)
    ut4 = user_table.reshape(-1, W)
    pt4 = product_table.reshape(-1, W)
    it4 = interaction_table.reshape(-1, W)
    u, p, i = _sc_gather(uid, pid, iid, ut4, pt4, it4)
    w1u4 = jnp.tile(W1[:D], (PACK, 1))
    w1p4 = jnp.tile(W1[D:2 * D], (PACK, 1))
    w1i4 = jnp.tile(W1[2 * D:], (PACK, 1))
    return _tc_mlp(u, p, i, uid.reshape(B, 1), pid.reshape(B, 1),
                   iid.reshape(B, 1), w1u4, w1p4, w1i4,
                   b1.reshape(1, H), W2, b2.reshape(1, 1))


# speedup vs baseline: 2.3902x; 2.3902x over previous
import functools

import jax
import jax.numpy as jnp
from jax import lax
from jax.experimental import pallas as pl
from jax.experimental.pallas import tpu as pltpu
from jax.experimental.pallas import tpu_sc as plsc

B = 16384
D = 32
H = 64
NU = 1000000
NP = 100000
NI = 1000
NC = 2
NS = 16
NW = NC * NS
L = 16
CH = 16384


def _chunks(n):
    nfull = n // CH
    out = [(k * CH, CH) for k in range(nfull - 1)]
    out.append(((nfull - 1) * CH, CH + n % CH))
    return out


def _add_base_and_gather(ids_hbm, base, scr, out_hbm, d, idx_v, col_v, sem_g):
    pltpu.sync_copy(ids_hbm, idx_v)

    def add_base(j, _):
        sl = pl.ds(j * L, L)
        idx_v[sl] = idx_v[sl] + base
        return _

    lax.fori_loop(0, B // L, add_base, 0, unroll=8)
    pltpu.async_copy(scr.at[idx_v], col_v, sem_g).wait()
    pltpu.sync_copy(col_v, out_hbm.at[d])


def _sc_gather_body(uid_hbm, pid_hbm, iid_hbm, ut_hbm, pt_hbm, itF_hbm,
                    u_out, p_out, i_out, uscr, pscr,
                    idx_v, col_v, buf0, buf1, ubuf_last, pbuf_last,
                    sem_r, sem_w0, sem_w1, sem_g):
    d = lax.axis_index("s") * NC + lax.axis_index("c")
    bufs = (buf0, buf1)
    wsems = (sem_w0, sem_w1)

    _add_base_and_gather(iid_hbm, d * NI, itF_hbm, i_out, d, idx_v, col_v,
                         sem_g)

    for ids_hbm, n_rows, tab_hbm, scr, out_hbm, last_buf in (
        (uid_hbm, NU, ut_hbm, uscr, u_out, ubuf_last),
        (pid_hbm, NP, pt_hbm, pscr, p_out, pbuf_last),
    ):
        base = d * n_rows
        chunks = _chunks(n_rows)
        writes = [None, None]
        for k, (off, sz) in enumerate(chunks):
            buf = last_buf if k == len(chunks) - 1 else bufs[k % 2]
            pltpu.async_copy(tab_hbm.at[d, pl.ds(off, sz)], buf,
                             sem_r).wait()
            if writes[k % 2] is not None:
                writes[k % 2].wait()
            writes[k % 2] = pltpu.async_copy(
                buf, scr.at[pl.ds(base + off, sz)], wsems[k % 2])
        for w in writes:
            if w is not None:
                w.wait()
        _add_base_and_gather(ids_hbm, base, scr, out_hbm, d, idx_v, col_v,
                             sem_g)


_sc_gather = pl.kernel(
    _sc_gather_body,
    out_type=(
        jax.ShapeDtypeStruct((NW, B), jnp.float32),
        jax.ShapeDtypeStruct((NW, B), jnp.float32),
        jax.ShapeDtypeStruct((NW, B), jnp.float32),
        jax.ShapeDtypeStruct((NW * NU,), jnp.float32),
        jax.ShapeDtypeStruct((NW * NP,), jnp.float32),
    ),
    mesh=plsc.VectorSubcoreMesh(core_axis_name="c", subcore_axis_name="s"),
    scratch_types=[
        pltpu.VMEM((B,), jnp.int32),
        pltpu.VMEM((B,), jnp.float32),
        pltpu.VMEM((CH,), jnp.float32),
        pltpu.VMEM((CH,), jnp.float32),
        pltpu.VMEM((CH + NU % CH,), jnp.float32),
        pltpu.VMEM((CH + NP % CH,), jnp.float32),
        pltpu.SemaphoreType.DMA,
        pltpu.SemaphoreType.DMA,
        pltpu.SemaphoreType.DMA,
        pltpu.SemaphoreType.DMA,
    ],
    compiler_params=pltpu.CompilerParams(use_tc_tiling_on_sc=True),
)


BS = 2048


def _mlp_body(u_ref, p_ref, i_ref, w1u_ref, w1p_ref, w1i_ref, b1_ref,
              w2_ref, b2_ref, out_ref):
    h = (jnp.dot(w1u_ref[...], u_ref[...], preferred_element_type=jnp.float32)
         + jnp.dot(w1p_ref[...], p_ref[...], preferred_element_type=jnp.float32)
         + jnp.dot(w1i_ref[...], i_ref[...], preferred_element_type=jnp.float32)
         + b1_ref[...])
    h = jnp.maximum(h, 0.0)
    out_ref[...] = (jnp.dot(w2_ref[...], h, preferred_element_type=jnp.float32)
                    + b2_ref[...])


def _tc_mlp(u, p, i, w1uT, w1pT, w1iT, b1c, w2T, b2c):
    grid = (B // BS,)
    emb_spec = pl.BlockSpec((D, BS), lambda j: (0, j))
    full = lambda shape: pl.BlockSpec(shape, lambda j: (0, 0))
    return pl.pallas_call(
        _mlp_body,
        grid=grid,
        in_specs=[emb_spec, emb_spec, emb_spec,
                  full((H, D)), full((H, D)), full((H, D)), full((H, 1)),
                  full((1, H)), full((1, 1))],
        out_specs=pl.BlockSpec((1, BS), lambda j: (0, j)),
        out_shape=jax.ShapeDtypeStruct((1, B), jnp.float32),
    )(u, p, i, w1uT, w1pT, w1iT, b1c, w2T, b2c)


def kernel(user_ids, product_ids, interaction_ids, user_table, product_table,
           interaction_table, W1, b1, W2, b2):
    uid = user_ids.astype(jnp.int32)
    pid = product_ids.astype(jnp.int32)
    iid = interaction_ids.astype(jnp.int32)
    u, p, i, _, _ = _sc_gather(uid, pid, iid, user_table.T, product_table.T,
                               interaction_table.T.reshape(-1))
    w1uT = W1[:D].T
    w1pT = W1[D:2 * D].T
    w1iT = W1[2 * D:].T
    outT = _tc_mlp(u, p, i, w1uT, w1pT, w1iT, b1.reshape(H, 1), W2.T,
                   b2.reshape(1, 1))
    return outT.reshape(B, 1)


# speedup vs baseline: 2.5761x; 1.0778x over previous
import functools

import jax
import jax.numpy as jnp
from jax import lax
from jax.experimental import pallas as pl
from jax.experimental.pallas import tpu as pltpu
from jax.experimental.pallas import tpu_sc as plsc

B = 16384
D = 32
H = 64
NU = 1000000
NP = 100000
NI = 1000
NC = 2
NS = 16
NW = NC * NS
L = 16
CH = 12800


def _chunks(n):
    nfull = n // CH
    out = [(k * CH, CH) for k in range(nfull - 1)]
    out.append(((nfull - 1) * CH, CH + n % CH))
    return out


def _add_base_and_gather(ids_hbm, base, scr, out_hbm, d, idx_v, col_v, sem_g):
    pltpu.sync_copy(ids_hbm, idx_v)

    def add_base(j, _):
        sl = pl.ds(j * L, L)
        idx_v[sl] = idx_v[sl] + base
        return _

    lax.fori_loop(0, B // L, add_base, 0, unroll=8)
    pltpu.async_copy(scr.at[idx_v], col_v, sem_g).wait()
    pltpu.sync_copy(col_v, out_hbm.at[d])


def _sc_gather_body(uid_hbm, pid_hbm, iid_hbm, ut_hbm, pt_hbm, itF_hbm,
                    u_out, p_out, i_out, uscr, pscr,
                    idx_v, col_v, buf0, buf1, buf2, buf3, ubuf_last, pbuf_last,
                    sem_r, sem_w0, sem_w1, sem_g):
    d = lax.axis_index("s") * NC + lax.axis_index("c")
    bufs = (buf0, buf1, buf2, buf3)
    wsems = (sem_w0, sem_w1)

    _add_base_and_gather(iid_hbm, d * NI, itF_hbm, i_out, d, idx_v, col_v,
                         sem_g)

    for ids_hbm, n_rows, tab_hbm, scr, out_hbm, last_buf in (
        (uid_hbm, NU, ut_hbm, uscr, u_out, ubuf_last),
        (pid_hbm, NP, pt_hbm, pscr, p_out, pbuf_last),
    ):
        base = d * n_rows
        chunks = _chunks(n_rows)
        n = len(chunks)
        bufmap = [last_buf if k == n - 1 else bufs[k % 4] for k in range(n)]
        reads = [None] * n
        writes = [None] * n
        w_waited = [False] * n

        def issue_read(k):
            off, sz = chunks[k]
            reads[k] = pltpu.async_copy(tab_hbm.at[d, pl.ds(off, sz)],
                                        bufmap[k], sem_r)

        issue_read(0)
        if n > 1:
            issue_read(1)
        for k in range(n):
            if k + 2 < n:
                if k - 2 >= 0 and not w_waited[k - 2]:
                    writes[k - 2].wait()
                    w_waited[k - 2] = True
                issue_read(k + 2)
            reads[k].wait()
            off, sz = chunks[k]
            writes[k] = pltpu.async_copy(bufmap[k],
                                         scr.at[pl.ds(base + off, sz)],
                                         wsems[k % 2])
        for k in range(n):
            if not w_waited[k]:
                writes[k].wait()
        _add_base_and_gather(ids_hbm, base, scr, out_hbm, d, idx_v, col_v,
                             sem_g)


_sc_gather = pl.kernel(
    _sc_gather_body,
    out_type=(
        jax.ShapeDtypeStruct((NW, B), jnp.float32),
        jax.ShapeDtypeStruct((NW, B), jnp.float32),
        jax.ShapeDtypeStruct((NW, B), jnp.float32),
        jax.ShapeDtypeStruct((NW * NU,), jnp.float32),
        jax.ShapeDtypeStruct((NW * NP,), jnp.float32),
    ),
    mesh=plsc.VectorSubcoreMesh(core_axis_name="c", subcore_axis_name="s"),
    scratch_types=[
        pltpu.VMEM((B,), jnp.int32),
        pltpu.VMEM((B,), jnp.float32),
        pltpu.VMEM((CH,), jnp.float32),
        pltpu.VMEM((CH,), jnp.float32),
        pltpu.VMEM((CH,), jnp.float32),
        pltpu.VMEM((CH,), jnp.float32),
        pltpu.VMEM((CH + NU % CH,), jnp.float32),
        pltpu.VMEM((CH + NP % CH,), jnp.float32),
        pltpu.SemaphoreType.DMA,
        pltpu.SemaphoreType.DMA,
        pltpu.SemaphoreType.DMA,
        pltpu.SemaphoreType.DMA,
    ],
    compiler_params=pltpu.CompilerParams(use_tc_tiling_on_sc=True),
)


BS = 2048


def _mlp_body(u_ref, p_ref, i_ref, w1u_ref, w1p_ref, w1i_ref, b1_ref,
              w2_ref, b2_ref, out_ref):
    h = (jnp.dot(w1u_ref[...], u_ref[...], preferred_element_type=jnp.float32)
         + jnp.dot(w1p_ref[...], p_ref[...], preferred_element_type=jnp.float32)
         + jnp.dot(w1i_ref[...], i_ref[...], preferred_element_type=jnp.float32)
         + b1_ref[...])
    h = jnp.maximum(h, 0.0)
    out_ref[...] = (jnp.dot(w2_ref[...], h, preferred_element_type=jnp.float32)
                    + b2_ref[...])


def _tc_mlp(u, p, i, w1uT, w1pT, w1iT, b1c, w2T, b2c):
    grid = (B // BS,)
    emb_spec = pl.BlockSpec((D, BS), lambda j: (0, j))
    full = lambda shape: pl.BlockSpec(shape, lambda j: (0, 0))
    return pl.pallas_call(
        _mlp_body,
        grid=grid,
        in_specs=[emb_spec, emb_spec, emb_spec,
                  full((H, D)), full((H, D)), full((H, D)), full((H, 1)),
                  full((1, H)), full((1, 1))],
        out_specs=pl.BlockSpec((1, BS), lambda j: (0, j)),
        out_shape=jax.ShapeDtypeStruct((1, B), jnp.float32),
    )(u, p, i, w1uT, w1pT, w1iT, b1c, w2T, b2c)


def kernel(user_ids, product_ids, interaction_ids, user_table, product_table,
           interaction_table, W1, b1, W2, b2):
    uid = user_ids.astype(jnp.int32)
    pid = product_ids.astype(jnp.int32)
    iid = interaction_ids.astype(jnp.int32)
    u, p, i, _, _ = _sc_gather(uid, pid, iid, user_table.T, product_table.T,
                               interaction_table.T.reshape(-1))
    w1uT = W1[:D].T
    w1pT = W1[D:2 * D].T
    w1iT = W1[2 * D:].T
    outT = _tc_mlp(u, p, i, w1uT, w1pT, w1iT, b1.reshape(H, 1), W2.T,
                   b2.reshape(1, 1))
    return outT.reshape(B, 1)
